# Initial kernel scaffold; baseline (speedup 1.0000x reference)
import jax
import jax.numpy as jnp
from jax.experimental import pallas as pl


def kernel(im1w_score):
    raise NotImplementedError("write your pallas kernel here")


# speedup vs baseline: 45.6657x; 45.6657x over previous
import numpy as np
import jax
import jax.numpy as jnp
from jax.experimental import pallas as pl

H = 512
W = 512
K = 512
BORDER = 8
NMS_R = 2
G_R = 3
G_TAPS = [float(np.exp(-2.0 * d * d, dtype=np.float64)) for d in range(G_R + 1)]


def _rfdet_kernel(x_ref, out_ref, tmask_ref, tv_ref):
    x = x_ref[0]

    ri = jax.lax.broadcasted_iota(jnp.int32, (H, W), 0)
    ci = jax.lax.broadcasted_iota(jnp.int32, (H, W), 1)
    inb = (ri >= BORDER) & (ri < H - BORDER) & (ci >= BORDER) & (ci < W - BORDER)
    x = jnp.where(inb, x, 0.0)

    cm = x
    for d in range(1, NMS_R + 1):
        cm = jnp.maximum(cm, jnp.maximum(jnp.roll(x, d, 1), jnp.roll(x, -d, 1)))
    mv = cm
    for d in range(1, NMS_R + 1):
        mv = jnp.maximum(mv, jnp.maximum(jnp.roll(cm, d, 0), jnp.roll(cm, -d, 0)))
    s = jnp.where((x == mv) & (x > 0.0), x, 0.0)
    tv_ref[0] = s

    bits = jax.lax.bitcast_convert_type(s, jnp.int32)

    def body(_, carry):
        lo, hi = carry
        mid = jax.lax.div(lo + hi, jnp.int32(2))
        c = jnp.sum((bits >= mid).astype(jnp.int32))
        big = c >= K
        return jnp.where(big, mid, lo), jnp.where(big, hi, mid)

    thr, _ = jax.lax.fori_loop(
        0, 30, body, (jnp.int32(0), jnp.int32(1 << 30)), unroll=False
    )

    gt = bits > thr
    tie = bits == thr
    m = (K - jnp.sum(gt.astype(jnp.int32))).astype(jnp.float32)

    tf = tie.astype(jnp.float32)
    lower = (ci < ri).astype(jnp.float32)
    upper = (ri < ci).astype(jnp.float32)
    col_excl = jax.lax.dot(tf, upper, precision=jax.lax.Precision.HIGHEST)
    row_tot = jnp.sum(tf, axis=1, keepdims=True)
    row_excl = jax.lax.dot(lower, row_tot, precision=jax.lax.Precision.HIGHEST)
    rank = row_excl + col_excl

    tmask = gt | (tie & (rank < m))
    tmask_ref[0] = tmask

    sm = jnp.where(tmask, s, 0.0)
    tmp = G_TAPS[0] * sm
    for d in range(1, G_R + 1):
        tmp = tmp + G_TAPS[d] * (jnp.roll(sm, d, 1) + jnp.roll(sm, -d, 1))
    acc = G_TAPS[0] * tmp
    for d in range(1, G_R + 1):
        acc = acc + G_TAPS[d] * (jnp.roll(tmp, d, 0) + jnp.roll(tmp, -d, 0))
    out_ref[0] = jnp.clip(acc, 0.0, 1.0)


def kernel(im1w_score):
    x = im1w_score[:, :, :, 0]
    B = x.shape[0]
    out, tmask, tv = pl.pallas_call(
        _rfdet_kernel,
        grid=(B,),
        in_specs=[pl.BlockSpec((1, H, W), lambda b: (b, 0, 0))],
        out_specs=[
            pl.BlockSpec((1, H, W), lambda b: (b, 0, 0)),
            pl.BlockSpec((1, H, W), lambda b: (b, 0, 0)),
            pl.BlockSpec((1, H, W), lambda b: (b, 0, 0)),
        ],
        out_shape=[
            jax.ShapeDtypeStruct((B, H, W), jnp.float32),
            jax.ShapeDtypeStruct((B, H, W), jnp.bool_),
            jax.ShapeDtypeStruct((B, H, W), jnp.float32),
        ],
    )(x)
    return out[..., None], tmask[..., None], tv[..., None]


# speedup vs baseline: 51.6037x; 1.1300x over previous
import numpy as np
import jax
import jax.numpy as jnp
from jax.experimental import pallas as pl

H = 512
W = 512
K = 512
BORDER = 8
NMS_R = 2
G_R = 2
G_TAPS = [float(np.exp(-2.0 * d * d, dtype=np.float64)) for d in range(G_R + 1)]


def _rfdet_kernel(x_ref, out_ref, tmask_ref, tv_ref):
    x = x_ref[0]

    ri = jax.lax.broadcasted_iota(jnp.int32, (H, W), 0)
    ci = jax.lax.broadcasted_iota(jnp.int32, (H, W), 1)
    inb = (ri >= BORDER) & (ri < H - BORDER) & (ci >= BORDER) & (ci < W - BORDER)
    x = jnp.where(inb, x, 0.0)

    cm = x
    for d in range(1, NMS_R + 1):
        cm = jnp.maximum(cm, jnp.maximum(jnp.roll(x, d, 1), jnp.roll(x, -d, 1)))
    mv = cm
    for d in range(1, NMS_R + 1):
        mv = jnp.maximum(mv, jnp.maximum(jnp.roll(cm, d, 0), jnp.roll(cm, -d, 0)))
    s = jnp.where((x == mv) & (x > 0.0), x, 0.0)
    tv_ref[0] = s

    bits = jax.lax.bitcast_convert_type(s, jnp.int32)

    total = jnp.int32(H * W)

    def count_ge(mid):
        cnt = jnp.sum((bits >= mid).astype(jnp.int32))
        return jnp.where(mid <= 0, total, cnt)

    def body(_, carry):
        lo, hi = carry
        mid = jax.lax.div(lo + hi, jnp.int32(2))
        big = count_ge(mid) >= K
        return jnp.where(big, mid, lo), jnp.where(big, hi, mid)

    thr, _ = jax.lax.fori_loop(
        0, 30, body, (jnp.int32(0), jnp.int32(1 << 30))
    )

    c_ge = count_ge(thr)
    c_gt = count_ge(thr + 1)

    gt = bits > thr
    tie = bits == thr
    m = (K - c_gt).astype(jnp.float32)

    def no_rank():
        return jnp.zeros((H, W), jnp.float32)

    def tie_rank():
        tf = tie.astype(jnp.float32)
        lower = (ci < ri).astype(jnp.float32)
        upper = (ri < ci).astype(jnp.float32)
        col_excl = jax.lax.dot(tf, upper, precision=jax.lax.Precision.HIGHEST)
        row_tot = jnp.sum(tf, axis=1, keepdims=True)
        row_excl = jax.lax.dot(lower, row_tot, precision=jax.lax.Precision.HIGHEST)
        return row_excl + col_excl

    rank = jax.lax.cond(c_ge == K, no_rank, tie_rank)
    tmask = gt | (tie & (rank < m))
    tmask_ref[0] = tmask

    sm = jnp.where(tmask, s, 0.0)
    tmp = G_TAPS[0] * sm
    for d in range(1, G_R + 1):
        tmp = tmp + G_TAPS[d] * (jnp.roll(sm, d, 1) + jnp.roll(sm, -d, 1))
    acc = G_TAPS[0] * tmp
    for d in range(1, G_R + 1):
        acc = acc + G_TAPS[d] * (jnp.roll(tmp, d, 0) + jnp.roll(tmp, -d, 0))
    out_ref[0] = jnp.clip(acc, 0.0, 1.0)


def kernel(im1w_score):
    x = im1w_score[:, :, :, 0]
    B = x.shape[0]
    out, tmask, tv = pl.pallas_call(
        _rfdet_kernel,
        grid=(B,),
        in_specs=[pl.BlockSpec((1, H, W), lambda b: (b, 0, 0))],
        out_specs=[
            pl.BlockSpec((1, H, W), lambda b: (b, 0, 0)),
            pl.BlockSpec((1, H, W), lambda b: (b, 0, 0)),
            pl.BlockSpec((1, H, W), lambda b: (b, 0, 0)),
        ],
        out_shape=[
            jax.ShapeDtypeStruct((B, H, W), jnp.float32),
            jax.ShapeDtypeStruct((B, H, W), jnp.bool_),
            jax.ShapeDtypeStruct((B, H, W), jnp.float32),
        ],
    )(x)
    return out[..., None], tmask[..., None], tv[..., None]


# speedup vs baseline: 66.2249x; 1.2833x over previous
import numpy as np
import jax
import jax.numpy as jnp
from jax.experimental import pallas as pl

H = 512
W = 512
K = 512
BORDER = 8
NMS_R = 2
G_R = 2
G_TAPS = [float(np.exp(-2.0 * d * d, dtype=np.float64)) for d in range(G_R + 1)]


def _rfdet_kernel(x_ref, out_ref, tmask_ref, tv_ref):
    x = x_ref[0].reshape(H, W)

    ri = jax.lax.broadcasted_iota(jnp.int32, (H, W), 0)
    ci = jax.lax.broadcasted_iota(jnp.int32, (H, W), 1)
    inb = (ri >= BORDER) & (ri < H - BORDER) & (ci >= BORDER) & (ci < W - BORDER)
    x = jnp.where(inb, x, 0.0)

    cm = x
    for d in range(1, NMS_R + 1):
        cm = jnp.maximum(cm, jnp.maximum(jnp.roll(x, d, 1), jnp.roll(x, -d, 1)))
    mv = cm
    for d in range(1, NMS_R + 1):
        mv = jnp.maximum(mv, jnp.maximum(jnp.roll(cm, d, 0), jnp.roll(cm, -d, 0)))
    s = jnp.where((x == mv) & (x > 0.0), x, 0.0)
    tv_ref[0] = s.reshape(H // 2, 8, 128)

    bits = jax.lax.bitcast_convert_type(s, jnp.int32)

    total = jnp.int32(H * W)

    def count_ge(mid):
        cnt = jnp.sum((bits >= mid).astype(jnp.int32))
        return jnp.where(mid <= 0, total, cnt)

    def body(_, carry):
        lo, hi = carry
        mid = jax.lax.div(lo + hi, jnp.int32(2))
        big = count_ge(mid) >= K
        return jnp.where(big, mid, lo), jnp.where(big, hi, mid)

    thr, _ = jax.lax.fori_loop(
        0, 30, body, (jnp.int32(0), jnp.int32(1 << 30))
    )

    c_ge = count_ge(thr)
    c_gt = count_ge(thr + 1)

    gt = bits > thr
    tie = bits == thr
    m = (K - c_gt).astype(jnp.float32)

    def no_rank():
        return jnp.zeros((H, W), jnp.float32)

    def tie_rank():
        tf = tie.astype(jnp.float32)
        lower = (ci < ri).astype(jnp.float32)
        upper = (ri < ci).astype(jnp.float32)
        col_excl = jax.lax.dot(tf, upper, precision=jax.lax.Precision.HIGHEST)
        row_tot = jnp.sum(tf, axis=1, keepdims=True)
        row_excl = jax.lax.dot(lower, row_tot, precision=jax.lax.Precision.HIGHEST)
        return row_excl + col_excl

    rank = jax.lax.cond(c_ge == K, no_rank, tie_rank)
    tmask = gt | (tie & (rank < m))
    tmask_ref[0] = tmask.astype(jnp.int32).reshape(H // 2, 8, 128) > 0

    sm = jnp.where(tmask, s, 0.0)
    tmp = G_TAPS[0] * sm
    for d in range(1, G_R + 1):
        tmp = tmp + G_TAPS[d] * (jnp.roll(sm, d, 1) + jnp.roll(sm, -d, 1))
    acc = G_TAPS[0] * tmp
    for d in range(1, G_R + 1):
        acc = acc + G_TAPS[d] * (jnp.roll(tmp, d, 0) + jnp.roll(tmp, -d, 0))
    out_ref[0] = jnp.clip(acc, 0.0, 1.0).reshape(H // 2, 8, 128)


def kernel(im1w_score):
    B = im1w_score.shape[0]
    x = jnp.reshape(im1w_score, (B, H // 2, 8, 128))
    blk = pl.BlockSpec((1, H // 2, 8, 128), lambda b: (b, 0, 0, 0))
    out, tmask, tv = pl.pallas_call(
        _rfdet_kernel,
        grid=(B,),
        in_specs=[blk],
        out_specs=[blk, blk, blk],
        out_shape=[
            jax.ShapeDtypeStruct((B, H // 2, 8, 128), jnp.float32),
            jax.ShapeDtypeStruct((B, H // 2, 8, 128), jnp.bool_),
            jax.ShapeDtypeStruct((B, H // 2, 8, 128), jnp.float32),
        ],
    )(x)
    shape4 = (B, H, W, 1)
    return (jnp.reshape(out, shape4), jnp.reshape(tmask, shape4),
            jnp.reshape(tv, shape4))


# speedup vs baseline: 68.4958x; 1.0343x over previous
import numpy as np
import jax
import jax.numpy as jnp
from jax.experimental import pallas as pl

H = 512
W = 512
K = 512
BORDER = 8
NMS_R = 2
G_R = 2
G_TAPS = [float(np.exp(-2.0 * d * d, dtype=np.float64)) for d in range(G_R + 1)]


def _rfdet_kernel(x_ref, out_ref, tmask_ref, tv_ref):
    x = x_ref[0].reshape(H, W)

    ri = jax.lax.broadcasted_iota(jnp.int32, (H, W), 0)
    ci = jax.lax.broadcasted_iota(jnp.int32, (H, W), 1)
    inb = (ri >= BORDER) & (ri < H - BORDER) & (ci >= BORDER) & (ci < W - BORDER)
    x = jnp.where(inb, x, 0.0)

    cm = x
    for d in range(1, NMS_R + 1):
        cm = jnp.maximum(cm, jnp.maximum(jnp.roll(x, d, 1), jnp.roll(x, -d, 1)))
    mv = cm
    for d in range(1, NMS_R + 1):
        mv = jnp.maximum(mv, jnp.maximum(jnp.roll(cm, d, 0), jnp.roll(cm, -d, 0)))
    s = jnp.where((x == mv) & (x > 0.0), x, 0.0)
    tv_ref[0] = s.reshape(H // 2, 8, 128)

    bits = jax.lax.bitcast_convert_type(s, jnp.int32)

    q = jnp.maximum(s, jnp.roll(s, -1, 0))
    q2 = jnp.maximum(q, jnp.roll(q, -1, 1))
    hri = jax.lax.broadcasted_iota(jnp.int32, (H // 2, W), 0)
    hci = jax.lax.broadcasted_iota(jnp.int32, (H // 2, W), 1)
    sel_rows = (hci == 2 * hri).astype(jnp.float32)
    vri = jax.lax.broadcasted_iota(jnp.int32, (W, W // 2), 0)
    vci = jax.lax.broadcasted_iota(jnp.int32, (W, W // 2), 1)
    sel_cols = (vri == 2 * vci).astype(jnp.float32)
    red = jax.lax.dot(
        jax.lax.dot(sel_rows, q2, precision=jax.lax.Precision.HIGHEST),
        sel_cols,
        precision=jax.lax.Precision.HIGHEST,
    )
    rbits = jax.lax.bitcast_convert_type(red, jnp.int32)

    total = jnp.int32(H * W)
    n_full = jnp.sum((bits > 0).astype(jnp.int32))
    n_red = jnp.sum((rbits > 0).astype(jnp.int32))
    multi = n_full != n_red

    def count_ge(mid):
        cnt = jax.lax.cond(
            multi,
            lambda: jnp.sum((bits >= mid).astype(jnp.int32)),
            lambda: jnp.sum((rbits >= mid).astype(jnp.int32)),
        )
        return jnp.where(mid <= 0, total, cnt)

    def body(_, carry):
        lo, hi = carry
        mid = jax.lax.div(lo + hi, jnp.int32(2))
        big = count_ge(mid) >= K
        return jnp.where(big, mid, lo), jnp.where(big, hi, mid)

    thr, _ = jax.lax.fori_loop(
        0, 30, body, (jnp.int32(0), jnp.int32(1 << 30))
    )

    c_ge = count_ge(thr)
    c_gt = count_ge(thr + 1)

    gt = bits > thr
    tie = bits == thr
    m = (K - c_gt).astype(jnp.float32)

    def no_rank():
        return jnp.zeros((H, W), jnp.float32)

    def tie_rank():
        tf = tie.astype(jnp.float32)
        lower = (ci < ri).astype(jnp.float32)
        upper = (ri < ci).astype(jnp.float32)
        col_excl = jax.lax.dot(tf, upper, precision=jax.lax.Precision.HIGHEST)
        row_tot = jnp.sum(tf, axis=1, keepdims=True)
        row_excl = jax.lax.dot(lower, row_tot, precision=jax.lax.Precision.HIGHEST)
        return row_excl + col_excl

    rank = jax.lax.cond(c_ge == K, no_rank, tie_rank)
    tmask = gt | (tie & (rank < m))
    tmask_ref[0] = tmask.astype(jnp.int32).reshape(H // 2, 8, 128) > 0

    sm = jnp.where(tmask, s, 0.0)
    tmp = G_TAPS[0] * sm
    for d in range(1, G_R + 1):
        tmp = tmp + G_TAPS[d] * (jnp.roll(sm, d, 1) + jnp.roll(sm, -d, 1))
    acc = G_TAPS[0] * tmp
    for d in range(1, G_R + 1):
        acc = acc + G_TAPS[d] * (jnp.roll(tmp, d, 0) + jnp.roll(tmp, -d, 0))
    out_ref[0] = jnp.clip(acc, 0.0, 1.0).reshape(H // 2, 8, 128)


def kernel(im1w_score):
    B = im1w_score.shape[0]
    x = jnp.reshape(im1w_score, (B, H // 2, 8, 128))
    blk = pl.BlockSpec((1, H // 2, 8, 128), lambda b: (b, 0, 0, 0))
    out, tmask, tv = pl.pallas_call(
        _rfdet_kernel,
        grid=(B,),
        in_specs=[blk],
        out_specs=[blk, blk, blk],
        out_shape=[
            jax.ShapeDtypeStruct((B, H // 2, 8, 128), jnp.float32),
            jax.ShapeDtypeStruct((B, H // 2, 8, 128), jnp.bool_),
            jax.ShapeDtypeStruct((B, H // 2, 8, 128), jnp.float32),
        ],
    )(x)
    shape4 = (B, H, W, 1)
    return (jnp.reshape(out, shape4), jnp.reshape(tmask, shape4),
            jnp.reshape(tv, shape4))


# speedup vs baseline: 77.5257x; 1.1318x over previous
import numpy as np
import jax
import jax.numpy as jnp
from jax.experimental import pallas as pl

H = 512
W = 512
K = 512
BORDER = 8
NMS_R = 2
G_R = 2
G_TAPS = [float(np.exp(-2.0 * d * d, dtype=np.float64)) for d in range(G_R + 1)]


def _rfdet_kernel(x_ref, out_ref, tmask_ref, tv_ref):
    x = x_ref[0].reshape(H, W)

    ri = jax.lax.broadcasted_iota(jnp.int32, (H, W), 0)
    ci = jax.lax.broadcasted_iota(jnp.int32, (H, W), 1)
    inb = (ri >= BORDER) & (ri < H - BORDER) & (ci >= BORDER) & (ci < W - BORDER)
    x = jnp.where(inb, x, 0.0)

    cm = x
    for d in range(1, NMS_R + 1):
        cm = jnp.maximum(cm, jnp.maximum(jnp.roll(x, d, 1), jnp.roll(x, -d, 1)))
    mv = cm
    for d in range(1, NMS_R + 1):
        mv = jnp.maximum(mv, jnp.maximum(jnp.roll(cm, d, 0), jnp.roll(cm, -d, 0)))
    s = jnp.where((x == mv) & (x > 0.0), x, 0.0)
    tv_ref[0] = s.reshape(H // 2, 8, 128)

    bits = jax.lax.bitcast_convert_type(s, jnp.int32)

    q = jnp.maximum(s, jnp.roll(s, -1, 0))
    q2 = jnp.maximum(q, jnp.roll(q, -1, 1))
    hri = jax.lax.broadcasted_iota(jnp.int32, (H // 2, W), 0)
    hci = jax.lax.broadcasted_iota(jnp.int32, (H // 2, W), 1)
    sel_rows = (hci == 2 * hri).astype(jnp.float32)
    vri = jax.lax.broadcasted_iota(jnp.int32, (W, W // 2), 0)
    vci = jax.lax.broadcasted_iota(jnp.int32, (W, W // 2), 1)
    sel_cols = (vri == 2 * vci).astype(jnp.float32)
    red = jax.lax.dot(
        jax.lax.dot(sel_rows, q2, precision=jax.lax.Precision.HIGHEST),
        sel_cols,
        precision=jax.lax.Precision.HIGHEST,
    )
    rbits = jax.lax.bitcast_convert_type(red, jnp.int32)

    total = jnp.int32(H * W)
    n_full = jnp.sum((bits > 0).astype(jnp.int32))
    n_red = jnp.sum((rbits > 0).astype(jnp.int32))
    multi = n_full != n_red

    def count_ge(mid):
        cnt = jax.lax.cond(
            multi,
            lambda: jnp.sum((bits >= mid).astype(jnp.int32)),
            lambda: jnp.sum((rbits >= mid).astype(jnp.int32)),
        )
        return jnp.where(mid <= 0, total, cnt)

    def count7(arr, lo, step):
        t = jnp.int32(0)
        for tt in range(1, 8):
            c = jnp.sum((arr >= lo + tt * step).astype(jnp.int32))
            t = t + (c >= K).astype(jnp.int32)
        return t

    def body(_, carry):
        lo, w = carry
        step = jax.lax.shift_right_logical(w, 3)
        t = jax.lax.cond(
            multi,
            lambda: count7(bits, lo, step),
            lambda: count7(rbits, lo, step),
        )
        return lo + t * step, step

    thr, _ = jax.lax.fori_loop(
        0, 10, body, (jnp.int32(0), jnp.int32(1 << 30))
    )

    c_ge = count_ge(thr)
    c_gt = count_ge(thr + 1)

    gt = bits > thr
    tie = bits == thr
    m = (K - c_gt).astype(jnp.float32)

    def no_rank():
        return jnp.zeros((H, W), jnp.float32)

    def tie_rank():
        tf = tie.astype(jnp.float32)
        lower = (ci < ri).astype(jnp.float32)
        upper = (ri < ci).astype(jnp.float32)
        col_excl = jax.lax.dot(tf, upper, precision=jax.lax.Precision.HIGHEST)
        row_tot = jnp.sum(tf, axis=1, keepdims=True)
        row_excl = jax.lax.dot(lower, row_tot, precision=jax.lax.Precision.HIGHEST)
        return row_excl + col_excl

    rank = jax.lax.cond(c_ge == K, no_rank, tie_rank)
    tmask = gt | (tie & (rank < m))
    tmask_ref[0] = tmask.astype(jnp.int32).reshape(H // 2, 8, 128) > 0

    sm = jnp.where(tmask, s, 0.0)
    tmp = G_TAPS[0] * sm
    for d in range(1, G_R + 1):
        tmp = tmp + G_TAPS[d] * (jnp.roll(sm, d, 1) + jnp.roll(sm, -d, 1))
    acc = G_TAPS[0] * tmp
    for d in range(1, G_R + 1):
        acc = acc + G_TAPS[d] * (jnp.roll(tmp, d, 0) + jnp.roll(tmp, -d, 0))
    out_ref[0] = jnp.clip(acc, 0.0, 1.0).reshape(H // 2, 8, 128)


def kernel(im1w_score):
    B = im1w_score.shape[0]
    x = jnp.reshape(im1w_score, (B, H // 2, 8, 128))
    blk = pl.BlockSpec((1, H // 2, 8, 128), lambda b: (b, 0, 0, 0))
    out, tmask, tv = pl.pallas_call(
        _rfdet_kernel,
        grid=(B,),
        in_specs=[blk],
        out_specs=[blk, blk, blk],
        out_shape=[
            jax.ShapeDtypeStruct((B, H // 2, 8, 128), jnp.float32),
            jax.ShapeDtypeStruct((B, H // 2, 8, 128), jnp.bool_),
            jax.ShapeDtypeStruct((B, H // 2, 8, 128), jnp.float32),
        ],
    )(x)
    shape4 = (B, H, W, 1)
    return (jnp.reshape(out, shape4), jnp.reshape(tmask, shape4),
            jnp.reshape(tv, shape4))


# speedup vs baseline: 77.9597x; 1.0056x over previous
import numpy as np
import jax
import jax.numpy as jnp
from jax.experimental import pallas as pl

H = 512
W = 512
K = 512
BORDER = 8
NMS_R = 2
G_R = 2
G_TAPS = [float(np.exp(-2.0 * d * d, dtype=np.float64)) for d in range(G_R + 1)]


def _rfdet_kernel(x_ref, out_ref, tmask_ref, tv_ref):
    x = x_ref[0].reshape(H, W)

    ri = jax.lax.broadcasted_iota(jnp.int32, (H, W), 0)
    ci = jax.lax.broadcasted_iota(jnp.int32, (H, W), 1)
    inb = (ri >= BORDER) & (ri < H - BORDER) & (ci >= BORDER) & (ci < W - BORDER)
    x = jnp.where(inb, x, 0.0)

    cm = x
    for d in range(1, NMS_R + 1):
        cm = jnp.maximum(cm, jnp.maximum(jnp.roll(x, d, 1), jnp.roll(x, -d, 1)))
    mv = cm
    for d in range(1, NMS_R + 1):
        mv = jnp.maximum(mv, jnp.maximum(jnp.roll(cm, d, 0), jnp.roll(cm, -d, 0)))
    s = jnp.where((x == mv) & (x > 0.0), x, 0.0)
    tv_ref[0] = s.reshape(H // 2, 8, 128)

    bits = jax.lax.bitcast_convert_type(s, jnp.int32)

    q = jnp.maximum(s, jnp.roll(s, -1, 0))
    q2 = jnp.maximum(q, jnp.roll(q, -1, 1))
    hri = jax.lax.broadcasted_iota(jnp.int32, (H // 2, W), 0)
    hci = jax.lax.broadcasted_iota(jnp.int32, (H // 2, W), 1)
    sel_rows = (hci == 2 * hri).astype(jnp.float32)
    vri = jax.lax.broadcasted_iota(jnp.int32, (W, W // 2), 0)
    vci = jax.lax.broadcasted_iota(jnp.int32, (W, W // 2), 1)
    sel_cols = (vri == 2 * vci).astype(jnp.float32)
    red = jax.lax.dot(
        jax.lax.dot(sel_rows, q2, precision=jax.lax.Precision.HIGHEST),
        sel_cols,
        precision=jax.lax.Precision.HIGHEST,
    )
    rbits = jax.lax.bitcast_convert_type(red, jnp.int32)

    total = jnp.int32(H * W)
    n_full = jnp.sum((bits > 0).astype(jnp.int32))
    n_red = jnp.sum((rbits > 0).astype(jnp.int32))
    multi = n_full != n_red

    def count_ge(mid):
        cnt = jax.lax.cond(
            multi,
            lambda: jnp.sum((bits >= mid).astype(jnp.int32)),
            lambda: jnp.sum((rbits >= mid).astype(jnp.int32)),
        )
        return jnp.where(mid <= 0, total, cnt)

    def count7(arr, lo, step):
        t = jnp.int32(0)
        for tt in range(1, 8):
            c = jnp.sum((arr >= lo + tt * step).astype(jnp.int32))
            t = t + (c >= K).astype(jnp.int32)
        return t

    thr = jnp.int32(0)
    for it in range(10):
        step = jnp.int32(1 << (27 - 3 * it))
        t = jax.lax.cond(
            multi,
            lambda lo=thr, st=step: count7(bits, lo, st),
            lambda lo=thr, st=step: count7(rbits, lo, st),
        )
        thr = thr + t * step

    c_ge = count_ge(thr)
    c_gt = count_ge(thr + 1)

    gt = bits > thr
    tie = bits == thr
    m = (K - c_gt).astype(jnp.float32)

    def no_rank():
        return jnp.zeros((H, W), jnp.float32)

    def tie_rank():
        tf = tie.astype(jnp.float32)
        lower = (ci < ri).astype(jnp.float32)
        upper = (ri < ci).astype(jnp.float32)
        col_excl = jax.lax.dot(tf, upper, precision=jax.lax.Precision.HIGHEST)
        row_tot = jnp.sum(tf, axis=1, keepdims=True)
        row_excl = jax.lax.dot(lower, row_tot, precision=jax.lax.Precision.HIGHEST)
        return row_excl + col_excl

    rank = jax.lax.cond(c_ge == K, no_rank, tie_rank)
    tmask = gt | (tie & (rank < m))
    tmask_ref[0] = tmask.astype(jnp.int32).reshape(H // 2, 8, 128) > 0

    sm = jnp.where(tmask, s, 0.0)
    tmp = G_TAPS[0] * sm
    for d in range(1, G_R + 1):
        tmp = tmp + G_TAPS[d] * (jnp.roll(sm, d, 1) + jnp.roll(sm, -d, 1))
    acc = G_TAPS[0] * tmp
    for d in range(1, G_R + 1):
        acc = acc + G_TAPS[d] * (jnp.roll(tmp, d, 0) + jnp.roll(tmp, -d, 0))
    out_ref[0] = jnp.clip(acc, 0.0, 1.0).reshape(H // 2, 8, 128)


def kernel(im1w_score):
    B = im1w_score.shape[0]
    x = jnp.reshape(im1w_score, (B, H // 2, 8, 128))
    blk = pl.BlockSpec((1, H // 2, 8, 128), lambda b: (b, 0, 0, 0))
    out, tmask, tv = pl.pallas_call(
        _rfdet_kernel,
        grid=(B,),
        in_specs=[blk],
        out_specs=[blk, blk, blk],
        out_shape=[
            jax.ShapeDtypeStruct((B, H // 2, 8, 128), jnp.float32),
            jax.ShapeDtypeStruct((B, H // 2, 8, 128), jnp.bool_),
            jax.ShapeDtypeStruct((B, H // 2, 8, 128), jnp.float32),
        ],
    )(x)
    shape4 = (B, H, W, 1)
    return (jnp.reshape(out, shape4), jnp.reshape(tmask, shape4),
            jnp.reshape(tv, shape4))
